# baseline (speedup 1.0000x reference)
import jax
import jax.numpy as jnp
from jax import lax
from jax.experimental import pallas as pl
from jax.experimental.pallas import tpu as pltpu
from jax.experimental.pallas import tpu_sc as plsc

N_ROWS = 100000
N_COLS = 64
N_UPD = 16384

NC = 2
NS = 16
CHUNKS_PER_CORE = 4
R = 12544
LAST_BASE = N_ROWS - R
R_PAD = R + 8
TRASH = R

W = 128
UPD_PER_TILE = N_UPD // NS
WINDOWS_PER_TILE = UPD_PER_TILE // W

ROWS_PER_TILE = R // NS

_BASES = [0, R, 2 * R, 3 * R, 4 * R, 5 * R, 6 * R, LAST_BASE]
_HIS = [R, 2 * R, 3 * R, 4 * R, 5 * R, 6 * R, LAST_BASE, N_ROWS]


def _scatter_add_body(in_hbm, idx_hbm, val_hbm, out_hbm, acc, iw, vw):
    c = lax.axis_index("c")
    s = lax.axis_index("s")
    lanes = 16
    c_is_0 = c == 0

    for k in range(CHUNKS_PER_CORE):
        base = lax.select(c_is_0, _BASES[k], _BASES[4 + k])
        hi = lax.select(c_is_0, _HIS[k], _HIS[4 + k])
        base = pl.multiple_of(base, 8)

        r0 = s * ROWS_PER_TILE
        pltpu.sync_copy(in_hbm.at[pl.ds(base + r0, ROWS_PER_TILE), :],
                        acc.at[pl.ds(r0, ROWS_PER_TILE), pl.ds(0, N_COLS)])

        plsc.subcore_barrier()

        for w in range(WINDOWS_PER_TILE):
            u0 = s * UPD_PER_TILE + w * W
            pltpu.sync_copy(idx_hbm.at[pl.ds(u0, W)], iw.at[0])
            pltpu.sync_copy(val_hbm.at[pl.ds(u0, W), :],
                            vw.at[:, pl.ds(0, N_COLS)])
            for j in range(W // lanes):
                v = iw[0, pl.ds(j * lanes, lanes)]
                local = v - base
                in_range = (v >= base) & (v < hi)
                iw[0, pl.ds(j * lanes, lanes)] = jnp.where(in_range, local, TRASH)
            pltpu.sync_copy(vw, acc.at[iw.at[0]], add=True)

        plsc.subcore_barrier()

        pltpu.sync_copy(acc.at[pl.ds(r0, ROWS_PER_TILE), pl.ds(0, N_COLS)],
                        out_hbm.at[pl.ds(base + r0, ROWS_PER_TILE), :])

        plsc.subcore_barrier()


def kernel(input, index, value):
    orig_dtype = input.dtype
    inp = input.astype(jnp.int32)
    idx = index.astype(jnp.int32)
    val = value.astype(jnp.int32)

    mesh = plsc.VectorSubcoreMesh(core_axis_name="c", subcore_axis_name="s")
    f = pl.kernel(
        _scatter_add_body,
        out_type=jax.ShapeDtypeStruct((N_ROWS, N_COLS), jnp.int32),
        mesh=mesh,
        scratch_types=[
            pltpu.VMEM_SHARED((R_PAD, 2 * N_COLS), jnp.int32),
            pltpu.VMEM((1, W), jnp.int32),
            pltpu.VMEM((W, 2 * N_COLS), jnp.int32),
        ],
        compiler_params=pltpu.CompilerParams(use_tc_tiling_on_sc=False),
    )
    return f(inp, idx, val).astype(orig_dtype)


# speedup vs baseline: 1.4520x; 1.4520x over previous
import jax
import jax.numpy as jnp
from jax import lax
from jax.experimental import pallas as pl
from jax.experimental.pallas import tpu as pltpu
from jax.experimental.pallas import tpu_sc as plsc

N_ROWS = 100000
N_COLS = 64
N_UPD = 16384

NC = 2
NS = 16
CHUNKS_PER_CORE = 2
R = 25088
LAST_BASE = N_ROWS - R
R_PAD = R + 8
TRASH = R

W = 128
UPD_PER_TILE = N_UPD // NS
WINDOWS_PER_TILE = UPD_PER_TILE // W

ROWS_PER_TILE = R // NS

_BASES = [0, R, 2 * R, LAST_BASE]
_HIS = [R, 2 * R, LAST_BASE, N_ROWS]


def _scatter_add_body(in_hbm, idx_hbm, val_hbm, out_hbm,
                      acc, iwraw, iw, vw, sem):
    c = lax.axis_index("c")
    s = lax.axis_index("s")
    lanes = 16
    c_is_0 = c == 0

    pltpu.sync_copy(idx_hbm.at[pl.ds(s * UPD_PER_TILE, UPD_PER_TILE)],
                    iwraw.at[0])

    for k in range(CHUNKS_PER_CORE):
        base = lax.select(c_is_0, _BASES[k], _BASES[2 + k])
        hi = lax.select(c_is_0, _HIS[k], _HIS[2 + k])
        base = pl.multiple_of(base, 8)

        r0 = s * ROWS_PER_TILE
        pltpu.sync_copy(in_hbm.at[pl.ds(base + r0, ROWS_PER_TILE), :],
                        acc.at[pl.ds(r0, ROWS_PER_TILE), :])

        plsc.subcore_barrier()

        copies = [pltpu.make_async_copy(
            val_hbm.at[pl.ds(s * UPD_PER_TILE + w * W, W), :],
            vw.at[w % 2], sem.at[w % 2]) for w in range(WINDOWS_PER_TILE)]
        copies[0].start()
        for w in range(WINDOWS_PER_TILE):
            if w + 1 < WINDOWS_PER_TILE:
                copies[w + 1].start()
            for j in range(W // lanes):
                v = iwraw[0, pl.ds(w * W + j * lanes, lanes)]
                local = v - base
                in_range = (v >= base) & (v < hi)
                iw[0, pl.ds(j * lanes, lanes)] = jnp.where(in_range, local, TRASH)
            copies[w].wait()
            pltpu.sync_copy(vw.at[w % 2], acc.at[iw.at[0]], add=True)

        plsc.subcore_barrier()

        pltpu.sync_copy(acc.at[pl.ds(r0, ROWS_PER_TILE), :],
                        out_hbm.at[pl.ds(base + r0, ROWS_PER_TILE), :])

        plsc.subcore_barrier()


def kernel(input, index, value):
    mesh = plsc.VectorSubcoreMesh(core_axis_name="c", subcore_axis_name="s")
    f = pl.kernel(
        _scatter_add_body,
        out_type=jax.ShapeDtypeStruct((N_ROWS, N_COLS), jnp.int32),
        mesh=mesh,
        scratch_types=[
            pltpu.VMEM_SHARED((R_PAD, N_COLS), jnp.int32),
            pltpu.VMEM((1, UPD_PER_TILE), jnp.int32),
            pltpu.VMEM((1, W), jnp.int32),
            pltpu.VMEM((2, W, N_COLS), jnp.int32),
            pltpu.SemaphoreType.DMA((2,)),
        ],
        compiler_params=pltpu.CompilerParams(use_tc_tiling_on_sc=False),
    )
    return f(input.astype(jnp.int32), index.astype(jnp.int32),
             value.astype(jnp.int32)).astype(input.dtype)


# speedup vs baseline: 1.4984x; 1.0320x over previous
import jax
import jax.numpy as jnp
from jax import lax
from jax.experimental import pallas as pl
from jax.experimental.pallas import tpu as pltpu
from jax.experimental.pallas import tpu_sc as plsc

N_ROWS = 100000
N_COLS = 64
N_UPD = 16384

NC = 2
NS = 16
CHUNKS_PER_CORE = 2
R = 25088
LAST_BASE = N_ROWS - R
R_PAD = R + 8
TRASH = R

W = 128
UPD_PER_TILE = N_UPD // NS
WINDOWS_PER_TILE = UPD_PER_TILE // W

ROWS_PER_TILE = R // NS

_BASES = [0, R, 2 * R, LAST_BASE]
_HIS = [R, 2 * R, LAST_BASE, N_ROWS]


def _scatter_add_body(in_hbm, idx_hbm, val_hbm, out_hbm,
                      acc, iwraw, iw, vw, sem):
    c = lax.axis_index("c")
    s = lax.axis_index("s")
    lanes = 16
    c_is_0 = c == 0

    pltpu.sync_copy(idx_hbm.at[pl.ds(s * UPD_PER_TILE, UPD_PER_TILE)],
                    iwraw.at[0])

    for k in range(CHUNKS_PER_CORE):
        base = lax.select(c_is_0, _BASES[k], _BASES[2 + k])
        hi = lax.select(c_is_0, _HIS[k], _HIS[2 + k])
        base = pl.multiple_of(base, 8)

        r0 = s * ROWS_PER_TILE
        pltpu.sync_copy(in_hbm.at[pl.ds(base + r0, ROWS_PER_TILE), :],
                        acc.at[pl.ds(r0, ROWS_PER_TILE), :])

        plsc.subcore_barrier()

        copies = [pltpu.make_async_copy(
            val_hbm.at[pl.ds(s * UPD_PER_TILE + w * W, W), :],
            vw.at[w % 2], sem.at[w % 2]) for w in range(WINDOWS_PER_TILE)]
        copies[0].start()
        for w in range(WINDOWS_PER_TILE):
            if w + 1 < WINDOWS_PER_TILE:
                copies[w + 1].start()
            for j in range(W // lanes):
                v = iwraw[0, pl.ds(w * W + j * lanes, lanes)]
                local = v - base
                in_range = (v >= base) & (v < hi)
                iw[0, pl.ds(j * lanes, lanes)] = jnp.where(in_range, local, TRASH)
            copies[w].wait()
            pltpu.sync_copy(vw.at[w % 2], acc.at[iw.at[0]], add=True)

        plsc.subcore_barrier()

        pltpu.sync_copy(acc.at[pl.ds(r0, ROWS_PER_TILE), :],
                        out_hbm.at[pl.ds(base + r0, ROWS_PER_TILE), :])


def kernel(input, index, value):
    mesh = plsc.VectorSubcoreMesh(core_axis_name="c", subcore_axis_name="s")
    f = pl.kernel(
        _scatter_add_body,
        out_type=jax.ShapeDtypeStruct((N_ROWS, N_COLS), jnp.int32),
        mesh=mesh,
        scratch_types=[
            pltpu.VMEM_SHARED((R_PAD, N_COLS), jnp.int32),
            pltpu.VMEM((1, UPD_PER_TILE), jnp.int32),
            pltpu.VMEM((1, W), jnp.int32),
            pltpu.VMEM((2, W, N_COLS), jnp.int32),
            pltpu.SemaphoreType.DMA((2,)),
        ],
        compiler_params=pltpu.CompilerParams(use_tc_tiling_on_sc=False),
    )
    out = f(jnp.asarray(input, jnp.int32), jnp.asarray(index, jnp.int32),
            jnp.asarray(value, jnp.int32))
    return jnp.asarray(out, input.dtype)
